# Initial kernel scaffold; baseline (speedup 1.0000x reference)
import jax
import jax.numpy as jnp
from jax.experimental import pallas as pl


def kernel(Z, h, edge_index, edge_distance_feature, edge_distance_vec, edge_distance, edge_fc_W, edge_fc_b, msg_W1, msg_b1, msg_W2, msg_b2, vec_W1, vec_b1, vec_W2, vec_b2, sc_W1, sc_b1, sc_W2, sc_b2):
    raise NotImplementedError("write your pallas kernel here")


# speedup vs baseline: 10.3798x; 10.3798x over previous
import functools

import jax
import jax.numpy as jnp
from jax import lax
from jax.experimental import pallas as pl
from jax.experimental.pallas import tpu as pltpu
from jax.experimental.pallas import tpu_sc as plsc

N = 10000
E = 320000
VEC_IN = 16
SCALAR_IN = 128
EDGE_IN = 16
HID = 64
NH = 4
T = 5
SCALAR_OUT = 128
ROW = 128

NC = 2
NS = 16
NW = NC * NS
EW = E // NW
CH = 80
NCHUNK = EW // CH
NPS = 632
N_PAD = NPS * NS


def _silu(x):
    return x * (1.0 / (1.0 + jnp.exp(-x)))


HT = 256


def _sc_gather(ht, src, dst):
    mesh = plsc.VectorSubcoreMesh(core_axis_name="c", subcore_axis_name="s")

    @functools.partial(
        pl.kernel,
        out_type=(
            jax.ShapeDtypeStruct((E, SCALAR_IN), jnp.float32),
            jax.ShapeDtypeStruct((E, SCALAR_IN), jnp.float32),
            jax.ShapeDtypeStruct((E, 48), jnp.float32),
        ),
        mesh=mesh,
        scratch_types=[
            pltpu.VMEM((CH,), jnp.int32),
            pltpu.VMEM((CH,), jnp.int32),
            pltpu.VMEM((CH, HT), jnp.float32),
            pltpu.VMEM((CH, HT), jnp.float32),
            pltpu.VMEM((CH, 48), jnp.float32),
            pltpu.SemaphoreType.DMA,
            pltpu.SemaphoreType.DMA,
        ],
    )
    def k(ht_hbm, src_hbm, dst_hbm, hs_hbm, hd_hbm, zd_hbm,
          idx_s, idx_d, sb, db, zbuf, sem1, sem2):
        wid = lax.axis_index("s") * NC + lax.axis_index("c")

        @pl.loop(0, NCHUNK)
        def _(ci):
            base = wid * EW + ci * CH
            pltpu.sync_copy(src_hbm.at[pl.ds(base, CH)], idx_s)
            pltpu.sync_copy(dst_hbm.at[pl.ds(base, CH)], idx_d)
            cp_s = pltpu.async_copy(ht_hbm.at[idx_s], sb, sem1)
            cp_d = pltpu.async_copy(ht_hbm.at[idx_d], db, sem2)
            cp_s.wait()
            cp_d.wait()
            pltpu.sync_copy(sb.at[pl.ds(0, CH), pl.ds(0, SCALAR_IN)],
                            hs_hbm.at[pl.ds(base, CH)])
            pltpu.sync_copy(db.at[pl.ds(0, CH), pl.ds(0, SCALAR_IN)],
                            hd_hbm.at[pl.ds(base, CH)])

            @pl.loop(0, CH)
            def _(i):
                for j in range(3):
                    sl = (i, pl.ds(SCALAR_IN + j * 16, 16))
                    zbuf[i, pl.ds(j * 16, 16)] = db[sl] - sb[sl]

            pltpu.sync_copy(zbuf, zd_hbm.at[pl.ds(base, CH)])

    return k(ht, src, dst)


def _edge_body(hs_ref, hd_ref, zd_ref, edf_ref, ev_ref,
               efW_ref, efb_ref, w1hi_ref, w1hj_ref, w1in_ref, w1ef_ref,
               b1_ref, w2_ref, b2_ref, vw1_ref, vb1_ref, vw2_ref, vb2_ref,
               out_ref):
    be = hs_ref.shape[0]
    zd = zd_ref[...]
    ev = ev_ref[...]
    zfd = []
    for d in range(3):
        evd = jnp.broadcast_to(ev[:, d:d + 1], (be, 4))
        zfd.append(jnp.concatenate([zd[:, d * 16:(d + 1) * 16], evd], axis=1))
    inv = None
    for d in range(3):
        P = jnp.concatenate(
            [zfd[d][:, t * 4:(t + 1) * 4] for t in range(T) for _ in range(T)],
            axis=1)
        Q = jnp.concatenate([zfd[d]] * T, axis=1)
        inv = P * Q if inv is None else inv + P * Q
    n2 = jnp.sum(inv * inv, axis=1, keepdims=True)
    inv = inv / jnp.maximum(jnp.sqrt(n2), 1e-12)

    ef = jnp.dot(edf_ref[...], efW_ref[...],
                 preferred_element_type=jnp.float32) + efb_ref[...]
    pre = (jnp.dot(hd_ref[...], w1hi_ref[...], preferred_element_type=jnp.float32)
           + jnp.dot(hs_ref[...], w1hj_ref[...], preferred_element_type=jnp.float32)
           + jnp.dot(inv, w1in_ref[...], preferred_element_type=jnp.float32)
           + jnp.dot(ef, w1ef_ref[...], preferred_element_type=jnp.float32)
           + b1_ref[...])
    msg = _silu(pre)
    msg = _silu(jnp.dot(msg, w2_ref[...], preferred_element_type=jnp.float32)
                + b2_ref[...])
    bas = jnp.dot(_silu(jnp.dot(msg, vw1_ref[...],
                                preferred_element_type=jnp.float32) + vb1_ref[...]),
                  vw2_ref[...], preferred_element_type=jnp.float32) + vb2_ref[...]
    za = []
    for d in range(3):
        acc = None
        for t in range(T):
            tb = jnp.concatenate([zfd[d][:, t * 4:(t + 1) * 4]] * 4, axis=1)
            term = tb * bas[:, t * 16:(t + 1) * 16]
            acc = term if acc is None else acc + term
        za.append(acc)
    ones = jnp.ones((be, 16), jnp.float32)
    out_ref[...] = jnp.concatenate([msg] + za + [ones], axis=1)


def _tc_edge(hs, hd, zd, edf, ev, efW, efb, w1hi, w1hj, w1in, w1ef,
             b1, w2, b2, vw1, vb1, vw2, vb2, *, be=512, interpret=False):
    grid = (E // be,)
    row_spec = lambda w: pl.BlockSpec((be, w), lambda i: (i, 0))
    full = lambda a: pl.BlockSpec(a.shape, lambda i: (0,) * a.ndim)
    return pl.pallas_call(
        _edge_body,
        grid=grid,
        in_specs=[row_spec(SCALAR_IN), row_spec(SCALAR_IN), row_spec(48),
                  row_spec(EDGE_IN), row_spec(3),
                  full(efW), full(efb), full(w1hi), full(w1hj), full(w1in),
                  full(w1ef), full(b1), full(w2), full(b2), full(vw1),
                  full(vb1), full(vw2), full(vb2)],
        out_specs=row_spec(ROW),
        out_shape=jax.ShapeDtypeStruct((E, ROW), jnp.float32),
        interpret=interpret,
    )(hs, hd, zd, edf, ev, efW, efb, w1hi, w1hj, w1in, w1ef,
      b1, w2, b2, vw1, vb1, vw2, vb2)


def _sc_scatter(eo, dst):
    mesh = plsc.VectorSubcoreMesh(core_axis_name="c", subcore_axis_name="s")

    @functools.partial(
        pl.kernel,
        out_type=jax.ShapeDtypeStruct((NC, N_PAD, ROW), jnp.float32),
        mesh=mesh,
        scratch_types=[
            pltpu.VMEM((CH,), jnp.int32),
            pltpu.VMEM((CH, ROW), jnp.float32),
            pltpu.VMEM((8, ROW), jnp.float32),
            pltpu.VMEM_SHARED((N_PAD, ROW), jnp.float32),
            pltpu.SemaphoreType.DMA,
        ],
    )
    def k(eo_hbm, dst_hbm, part_hbm, idx_v, rows, zb, accum, sem):
        cid = lax.axis_index("c")
        sid = lax.axis_index("s")
        wid = sid * NC + cid

        @pl.loop(0, 8)
        def _(i):
            for j in range(ROW // 16):
                zb[i, pl.ds(j * 16, 16)] = jnp.zeros((16,), jnp.float32)

        @pl.loop(0, NPS // 8)
        def _(t):
            pltpu.sync_copy(zb, accum.at[pl.ds(sid * NPS + t * 8, 8)])

        plsc.subcore_barrier()

        @pl.loop(0, NCHUNK)
        def _(ci):
            base = wid * EW + ci * CH
            pltpu.sync_copy(dst_hbm.at[pl.ds(base, CH)], idx_v)
            pltpu.sync_copy(eo_hbm.at[pl.ds(base, CH)], rows)
            pltpu.sync_copy(rows, accum.at[idx_v], add=True)

        plsc.subcore_barrier()
        pltpu.sync_copy(accum.at[pl.ds(sid * NPS, NPS)],
                        part_hbm.at[cid, pl.ds(sid * NPS, NPS)])

    return k(eo, dst)


def _node_body(p0_ref, p1_ref, h_ref, wh_ref, wm_ref, b1_ref, w2_ref, b2_ref,
               z_ref, h_out_ref):
    p0 = p0_ref[...]
    p1 = p1_ref[...]
    m = p0[:, :HID] + p1[:, :HID]
    zsum = p0[:, HID:HID + 48] + p1[:, HID:HID + 48]
    cnt = p0[:, HID + 48:HID + 49] + p1[:, HID + 48:HID + 49]
    z_ref[...] = zsum / jnp.maximum(cnt, 1.0)
    pre = (jnp.dot(h_ref[...], wh_ref[...], preferred_element_type=jnp.float32)
           + jnp.dot(m, wm_ref[...], preferred_element_type=jnp.float32)
           + b1_ref[...])
    h_out_ref[...] = (jnp.dot(_silu(pre), w2_ref[...],
                              preferred_element_type=jnp.float32) + b2_ref[...])


def _tc_node(p0, p1, h, wh, wm, b1, w2, b2, *, bn=1000, interpret=False):
    grid = (N // bn,)
    row_spec = lambda w: pl.BlockSpec((bn, w), lambda i: (i, 0))
    full = lambda a: pl.BlockSpec(a.shape, lambda i: (0,) * a.ndim)
    return pl.pallas_call(
        _node_body,
        grid=grid,
        in_specs=[row_spec(ROW), row_spec(ROW), row_spec(SCALAR_IN),
                  full(wh), full(wm), full(b1), full(w2), full(b2)],
        out_specs=[row_spec(48), row_spec(SCALAR_IN)],
        out_shape=[jax.ShapeDtypeStruct((N, 48), jnp.float32),
                   jax.ShapeDtypeStruct((N, SCALAR_IN), jnp.float32)],
        interpret=interpret,
    )(p0, p1, h, wh, wm, b1, w2, b2)


def kernel(Z, h, edge_index, edge_distance_feature, edge_distance_vec,
           edge_distance, edge_fc_W, edge_fc_b, msg_W1, msg_b1, msg_W2, msg_b2,
           vec_W1, vec_b1, vec_W2, vec_b2, sc_W1, sc_b1, sc_W2, sc_b2):
    src = edge_index[0]
    dst = edge_index[1]
    zf = Z.reshape(N, 48)
    ht = jnp.concatenate([h, zf, jnp.zeros((N, HT - SCALAR_IN - 48), jnp.float32)],
                         axis=1)

    hs, hd, zd = _sc_gather(ht, src, dst)

    eo = _tc_edge(
        hs, hd, zd, edge_distance_feature, edge_distance_vec,
        edge_fc_W, edge_fc_b.reshape(1, HID),
        msg_W1[0:128], msg_W1[128:256], msg_W1[256:356], msg_W1[356:420],
        msg_b1.reshape(1, HID), msg_W2, msg_b2.reshape(1, HID),
        vec_W1, vec_b1.reshape(1, HID), vec_W2, vec_b2.reshape(1, 80))

    part = _sc_scatter(eo, dst)

    z_out, h_out = _tc_node(
        part[0, :N], part[1, :N], h,
        sc_W1[0:SCALAR_IN], sc_W1[SCALAR_IN:SCALAR_IN + HID],
        sc_b1.reshape(1, HID), sc_W2, sc_b2.reshape(1, SCALAR_OUT))

    return (z_out.reshape(N, 3, VEC_IN), h_out)


# speedup vs baseline: 31.9226x; 3.0755x over previous
import functools

import jax
import jax.numpy as jnp
from jax import lax
from jax.experimental import pallas as pl
from jax.experimental.pallas import tpu as pltpu
from jax.experimental.pallas import tpu_sc as plsc

N = 10000
E = 320000
VEC_IN = 16
SCALAR_IN = 128
EDGE_IN = 16
HID = 64
NH = 4
T = 5
SCALAR_OUT = 128
ROW = 128

NC = 2
NS = 16
NW = NC * NS
EW = E // NW
CH = 80
NCHUNK = EW // CH
NPS = 632
N_PAD = NPS * NS


def _silu(x):
    return x * (1.0 / (1.0 + jnp.exp(-x)))


HT = 256


def _sc_gather(ht, src, dst):
    mesh = plsc.VectorSubcoreMesh(core_axis_name="c", subcore_axis_name="s")

    @functools.partial(
        pl.kernel,
        out_type=(
            jax.ShapeDtypeStruct((E, SCALAR_IN), jnp.float32),
            jax.ShapeDtypeStruct((E, SCALAR_IN), jnp.float32),
            jax.ShapeDtypeStruct((E, 48), jnp.float32),
        ),
        mesh=mesh,
        scratch_types=[
            pltpu.VMEM((CH,), jnp.int32),
            pltpu.VMEM((CH,), jnp.int32),
            pltpu.VMEM((CH, HT), jnp.float32),
            pltpu.VMEM((CH, HT), jnp.float32),
            pltpu.VMEM((CH, 48), jnp.float32),
            pltpu.SemaphoreType.DMA,
            pltpu.SemaphoreType.DMA,
        ],
    )
    def k(ht_hbm, src_hbm, dst_hbm, hs_hbm, hd_hbm, zd_hbm,
          idx_s, idx_d, sb, db, zbuf, sem1, sem2):
        wid = lax.axis_index("s") * NC + lax.axis_index("c")

        @pl.loop(0, NCHUNK)
        def _(ci):
            base = wid * EW + ci * CH
            pltpu.sync_copy(src_hbm.at[pl.ds(base, CH)], idx_s)
            pltpu.sync_copy(dst_hbm.at[pl.ds(base, CH)], idx_d)
            cp_s = pltpu.async_copy(ht_hbm.at[idx_s], sb, sem1)
            cp_d = pltpu.async_copy(ht_hbm.at[idx_d], db, sem2)
            cp_s.wait()
            cp_d.wait()
            pltpu.sync_copy(sb.at[pl.ds(0, CH), pl.ds(0, SCALAR_IN)],
                            hs_hbm.at[pl.ds(base, CH)])
            pltpu.sync_copy(db.at[pl.ds(0, CH), pl.ds(0, SCALAR_IN)],
                            hd_hbm.at[pl.ds(base, CH)])

            @pl.loop(0, CH)
            def _(i):
                for j in range(3):
                    sl = (i, pl.ds(SCALAR_IN + j * 16, 16))
                    zbuf[i, pl.ds(j * 16, 16)] = db[sl] - sb[sl]

            pltpu.sync_copy(zbuf, zd_hbm.at[pl.ds(base, CH)])

    return k(ht, src, dst)


def _sel_mats():
    import numpy as np

    def zrow(d, t, h):
        return d * 16 + t * 4 + h if t < 4 else 48 + d

    Wp = np.zeros((51, 300), np.float32)
    Wq = np.zeros((51, 300), np.float32)
    Wr = np.zeros((300, 100), np.float32)
    for d in range(3):
        for t in range(T):
            for r in range(T):
                for h in range(NH):
                    c = d * 100 + (t * T + r) * NH + h
                    Wp[zrow(d, t, h), c] = 1.0
                    Wq[zrow(d, r, h), c] = 1.0
                    Wr[c, (t * T + r) * NH + h] = 1.0
    Wn = np.ones((100, 8), np.float32)
    Wa = np.zeros((51, 240), np.float32)
    Wb = np.zeros((80, 240), np.float32)
    Wz = np.zeros((240, 48), np.float32)
    for d in range(3):
        for t in range(T):
            for k in range(4):
                for h in range(NH):
                    c = d * 80 + t * 16 + k * 4 + h
                    Wa[zrow(d, t, h), c] = 1.0
                    Wb[t * 16 + k * 4 + h, c] = 1.0
                    Wz[c, d * 16 + k * 4 + h] = 1.0
    return tuple(jnp.asarray(m) for m in (Wp, Wq, Wr, Wn, Wa, Wb, Wz))


def _edge_body(hs_ref, hd_ref, zd_ref, edf_ref, ev_ref,
               efW_ref, efb_ref, w1hi_ref, w1hj_ref, w1in_ref, w1ef_ref,
               b1_ref, w2_ref, b2_ref, vw1_ref, vb1_ref, vw2_ref, vb2_ref,
               wp_ref, wq_ref, wr_ref, wn_ref, wa_ref, wb_ref, wz_ref,
               out_ref):
    be = hs_ref.shape[0]
    dot = lambda a, b: jnp.dot(a, b, preferred_element_type=jnp.float32)
    zcat = jnp.concatenate([zd_ref[...], ev_ref[...]], axis=1)
    P = dot(zcat, wp_ref[...])
    Q = dot(zcat, wq_ref[...])
    inv = dot(P * Q, wr_ref[...])
    n2 = dot(inv * inv, wn_ref[...])[:, :1]
    inv = inv / jnp.maximum(jnp.sqrt(n2), 1e-12)

    ef = jnp.dot(edf_ref[...], efW_ref[...],
                 preferred_element_type=jnp.float32) + efb_ref[...]
    pre = (jnp.dot(hd_ref[...], w1hi_ref[...], preferred_element_type=jnp.float32)
           + jnp.dot(hs_ref[...], w1hj_ref[...], preferred_element_type=jnp.float32)
           + jnp.dot(inv, w1in_ref[...], preferred_element_type=jnp.float32)
           + jnp.dot(ef, w1ef_ref[...], preferred_element_type=jnp.float32)
           + b1_ref[...])
    msg = _silu(pre)
    msg = _silu(jnp.dot(msg, w2_ref[...], preferred_element_type=jnp.float32)
                + b2_ref[...])
    bas = jnp.dot(_silu(jnp.dot(msg, vw1_ref[...],
                                preferred_element_type=jnp.float32) + vb1_ref[...]),
                  vw2_ref[...], preferred_element_type=jnp.float32) + vb2_ref[...]
    A = dot(zcat, wa_ref[...])
    B = dot(bas, wb_ref[...])
    za = dot(A * B, wz_ref[...])
    ones = jnp.ones((be, 16), jnp.float32)
    out_ref[...] = jnp.concatenate([msg, za, ones], axis=1)


def _tc_edge(hs, hd, zd, edf, ev, efW, efb, w1hi, w1hj, w1in, w1ef,
             b1, w2, b2, vw1, vb1, vw2, vb2, *, be=1280, interpret=False):
    grid = (E // be,)
    sel = _sel_mats()
    row_spec = lambda w: pl.BlockSpec((be, w), lambda i: (i, 0))
    full = lambda a: pl.BlockSpec(a.shape, lambda i: (0,) * a.ndim)
    consts = (efW, efb, w1hi, w1hj, w1in, w1ef, b1, w2, b2, vw1, vb1, vw2,
              vb2) + sel
    return pl.pallas_call(
        _edge_body,
        grid=grid,
        in_specs=[row_spec(SCALAR_IN), row_spec(SCALAR_IN), row_spec(48),
                  row_spec(EDGE_IN), row_spec(3)] + [full(c) for c in consts],
        out_specs=row_spec(ROW),
        out_shape=jax.ShapeDtypeStruct((E, ROW), jnp.float32),
        interpret=interpret,
    )(hs, hd, zd, edf, ev, *consts)


def _sc_scatter(eo, dst):
    mesh = plsc.VectorSubcoreMesh(core_axis_name="c", subcore_axis_name="s")

    @functools.partial(
        pl.kernel,
        out_type=jax.ShapeDtypeStruct((NC, N_PAD, ROW), jnp.float32),
        mesh=mesh,
        scratch_types=[
            pltpu.VMEM((CH,), jnp.int32),
            pltpu.VMEM((CH, ROW), jnp.float32),
            pltpu.VMEM((8, ROW), jnp.float32),
            pltpu.VMEM_SHARED((N_PAD, ROW), jnp.float32),
            pltpu.SemaphoreType.DMA,
        ],
    )
    def k(eo_hbm, dst_hbm, part_hbm, idx_v, rows, zb, accum, sem):
        cid = lax.axis_index("c")
        sid = lax.axis_index("s")
        wid = sid * NC + cid

        @pl.loop(0, 8)
        def _(i):
            for j in range(ROW // 16):
                zb[i, pl.ds(j * 16, 16)] = jnp.zeros((16,), jnp.float32)

        @pl.loop(0, NPS // 8)
        def _(t):
            pltpu.sync_copy(zb, accum.at[pl.ds(sid * NPS + t * 8, 8)])

        plsc.subcore_barrier()

        @pl.loop(0, NCHUNK)
        def _(ci):
            base = wid * EW + ci * CH
            pltpu.sync_copy(dst_hbm.at[pl.ds(base, CH)], idx_v)
            pltpu.sync_copy(eo_hbm.at[pl.ds(base, CH)], rows)
            pltpu.sync_copy(rows, accum.at[idx_v], add=True)

        plsc.subcore_barrier()
        pltpu.sync_copy(accum.at[pl.ds(sid * NPS, NPS)],
                        part_hbm.at[cid, pl.ds(sid * NPS, NPS)])

    return k(eo, dst)


def _node_body(p0_ref, p1_ref, h_ref, wh_ref, wm_ref, b1_ref, w2_ref, b2_ref,
               z_ref, h_out_ref):
    p0 = p0_ref[...]
    p1 = p1_ref[...]
    m = p0[:, :HID] + p1[:, :HID]
    zsum = p0[:, HID:HID + 48] + p1[:, HID:HID + 48]
    cnt = p0[:, HID + 48:HID + 49] + p1[:, HID + 48:HID + 49]
    z_ref[...] = zsum / jnp.maximum(cnt, 1.0)
    pre = (jnp.dot(h_ref[...], wh_ref[...], preferred_element_type=jnp.float32)
           + jnp.dot(m, wm_ref[...], preferred_element_type=jnp.float32)
           + b1_ref[...])
    h_out_ref[...] = (jnp.dot(_silu(pre), w2_ref[...],
                              preferred_element_type=jnp.float32) + b2_ref[...])


def _tc_node(p0, p1, h, wh, wm, b1, w2, b2, *, bn=1000, interpret=False):
    grid = (N // bn,)
    row_spec = lambda w: pl.BlockSpec((bn, w), lambda i: (i, 0))
    full = lambda a: pl.BlockSpec(a.shape, lambda i: (0,) * a.ndim)
    return pl.pallas_call(
        _node_body,
        grid=grid,
        in_specs=[row_spec(ROW), row_spec(ROW), row_spec(SCALAR_IN),
                  full(wh), full(wm), full(b1), full(w2), full(b2)],
        out_specs=[row_spec(48), row_spec(SCALAR_IN)],
        out_shape=[jax.ShapeDtypeStruct((N, 48), jnp.float32),
                   jax.ShapeDtypeStruct((N, SCALAR_IN), jnp.float32)],
        interpret=interpret,
    )(p0, p1, h, wh, wm, b1, w2, b2)


def kernel(Z, h, edge_index, edge_distance_feature, edge_distance_vec,
           edge_distance, edge_fc_W, edge_fc_b, msg_W1, msg_b1, msg_W2, msg_b2,
           vec_W1, vec_b1, vec_W2, vec_b2, sc_W1, sc_b1, sc_W2, sc_b2):
    src = edge_index[0]
    dst = edge_index[1]
    zf = Z.reshape(N, 48)
    ht = jnp.concatenate([h, zf, jnp.zeros((N, HT - SCALAR_IN - 48), jnp.float32)],
                         axis=1)

    hs, hd, zd = _sc_gather(ht, src, dst)

    eo = _tc_edge(
        hs, hd, zd, edge_distance_feature, edge_distance_vec,
        edge_fc_W, edge_fc_b.reshape(1, HID),
        msg_W1[0:128], msg_W1[128:256], msg_W1[256:356], msg_W1[356:420],
        msg_b1.reshape(1, HID), msg_W2, msg_b2.reshape(1, HID),
        vec_W1, vec_b1.reshape(1, HID), vec_W2, vec_b2.reshape(1, 80))

    part = _sc_scatter(eo, dst)

    z_out, h_out = _tc_node(
        part[0, :N], part[1, :N], h,
        sc_W1[0:SCALAR_IN], sc_W1[SCALAR_IN:SCALAR_IN + HID],
        sc_b1.reshape(1, HID), sc_W2, sc_b2.reshape(1, SCALAR_OUT))

    return (z_out.reshape(N, 3, VEC_IN), h_out)


# speedup vs baseline: 40.1104x; 1.2565x over previous
import functools

import jax
import jax.numpy as jnp
from jax import lax
from jax.experimental import pallas as pl
from jax.experimental.pallas import tpu as pltpu
from jax.experimental.pallas import tpu_sc as plsc

N = 10000
E = 320000
VEC_IN = 16
SCALAR_IN = 128
EDGE_IN = 16
HID = 64
NH = 4
T = 5
SCALAR_OUT = 128
ROW = 128

NC = 2
NS = 16
NW = NC * NS
EW = E // NW
CH = 80
NCHUNK = EW // CH
NPS = 632
N_PAD = NPS * NS


def _silu(x):
    return x * (1.0 / (1.0 + jnp.exp(-x)))


def _pre_body(h_ref, zf_ref, w1a_ref, w1b_ref, b1_ref, ts_ref, td_ref):
    bn = h_ref.shape[0]
    h = h_ref[...]
    zf = zf_ref[...]
    pad = jnp.zeros((bn, 16), jnp.float32)
    hb = jnp.dot(h, w1b_ref[...], preferred_element_type=jnp.float32)
    ha = jnp.dot(h, w1a_ref[...], preferred_element_type=jnp.float32) + b1_ref[...]
    ts_ref[...] = jnp.concatenate([hb, zf, pad], axis=1)
    td_ref[...] = jnp.concatenate([ha, zf, pad], axis=1)


def _tc_pre(h, zf, w1a, w1b, b1, *, bn=2000, interpret=False):
    row_spec = lambda w: pl.BlockSpec((bn, w), lambda i: (i, 0))
    full = lambda a: pl.BlockSpec(a.shape, lambda i: (0,) * a.ndim)
    return pl.pallas_call(
        _pre_body,
        grid=(N // bn,),
        in_specs=[row_spec(SCALAR_IN), row_spec(48), full(w1a), full(w1b),
                  full(b1)],
        out_specs=[row_spec(SCALAR_IN), row_spec(SCALAR_IN)],
        out_shape=[jax.ShapeDtypeStruct((N, SCALAR_IN), jnp.float32),
                   jax.ShapeDtypeStruct((N, SCALAR_IN), jnp.float32)],
        interpret=interpret,
    )(h, zf, w1a, w1b, b1)


def _sc_gather(ts, td, src, dst):
    mesh = plsc.VectorSubcoreMesh(core_axis_name="c", subcore_axis_name="s")

    @functools.partial(
        pl.kernel,
        out_type=(
            jax.ShapeDtypeStruct((E, SCALAR_IN), jnp.float32),
            jax.ShapeDtypeStruct((E, SCALAR_IN), jnp.float32),
        ),
        mesh=mesh,
        scratch_types=[
            pltpu.VMEM((CH,), jnp.int32),
            pltpu.VMEM((CH,), jnp.int32),
            pltpu.VMEM((CH, SCALAR_IN), jnp.float32),
            pltpu.VMEM((CH, SCALAR_IN), jnp.float32),
            pltpu.SemaphoreType.DMA,
            pltpu.SemaphoreType.DMA,
        ],
    )
    def k(ts_hbm, td_hbm, src_hbm, dst_hbm, gs_hbm, gd_hbm,
          idx_s, idx_d, sb, db, sem1, sem2):
        wid = lax.axis_index("s") * NC + lax.axis_index("c")

        @pl.loop(0, NCHUNK)
        def _(ci):
            base = wid * EW + ci * CH
            pltpu.sync_copy(src_hbm.at[pl.ds(base, CH)], idx_s)
            pltpu.sync_copy(dst_hbm.at[pl.ds(base, CH)], idx_d)
            cp_s = pltpu.async_copy(ts_hbm.at[idx_s], sb, sem1)
            cp_d = pltpu.async_copy(td_hbm.at[idx_d], db, sem2)
            cp_s.wait()
            cp_d.wait()
            pltpu.sync_copy(sb, gs_hbm.at[pl.ds(base, CH)])
            pltpu.sync_copy(db, gd_hbm.at[pl.ds(base, CH)])

    return k(ts, td, src, dst)


def _sel_mats():
    import numpy as np

    def zrow(d, t, h):
        return d * 16 + t * 4 + h if t < 4 else 48 + d

    Wp = np.zeros((51, 300), np.float32)
    Wq = np.zeros((51, 300), np.float32)
    Wr = np.zeros((300, 100), np.float32)
    for d in range(3):
        for t in range(T):
            for r in range(T):
                for h in range(NH):
                    c = d * 100 + (t * T + r) * NH + h
                    Wp[zrow(d, t, h), c] = 1.0
                    Wq[zrow(d, r, h), c] = 1.0
                    Wr[c, (t * T + r) * NH + h] = 1.0
    Wn = np.ones((100, 8), np.float32)
    Wa = np.zeros((51, 240), np.float32)
    Wb = np.zeros((80, 240), np.float32)
    Wz = np.zeros((240, 48), np.float32)
    for d in range(3):
        for t in range(T):
            for k in range(4):
                for h in range(NH):
                    c = d * 80 + t * 16 + k * 4 + h
                    Wa[zrow(d, t, h), c] = 1.0
                    Wb[t * 16 + k * 4 + h, c] = 1.0
                    Wz[c, d * 16 + k * 4 + h] = 1.0
    return tuple(jnp.asarray(m) for m in (Wp, Wq, Wr, Wn, Wa, Wb, Wz))


def _edge_body(gs_ref, gd_ref, edf_ref, ev_ref,
               efW_ref, efb_ref, w1in_ref, w1ef_ref,
               w2_ref, b2_ref, vw1_ref, vb1_ref, vw2_ref, vb2_ref,
               wp_ref, wq_ref, wr_ref, wn_ref, wa_ref, wb_ref, wz_ref,
               out_ref):
    dot = lambda a, b: jnp.dot(a, b, preferred_element_type=jnp.float32)
    gs = gs_ref[...]
    gd = gd_ref[...]
    zdiff = gd[:, HID:HID + 48] - gs[:, HID:HID + 48]
    zcat = jnp.concatenate([zdiff, ev_ref[...]], axis=1)
    P = dot(zcat, wp_ref[...])
    Q = dot(zcat, wq_ref[...])
    inv = dot(P * Q, wr_ref[...])
    n2 = dot(inv * inv, wn_ref[...])[:, :1]
    inv = inv / jnp.maximum(jnp.sqrt(n2), 1e-12)

    ef = jnp.dot(edf_ref[...], efW_ref[...],
                 preferred_element_type=jnp.float32) + efb_ref[...]
    pre = (gd[:, :HID] + gs[:, :HID]
           + jnp.dot(inv, w1in_ref[...], preferred_element_type=jnp.float32)
           + jnp.dot(ef, w1ef_ref[...], preferred_element_type=jnp.float32))
    msg = _silu(pre)
    msg = _silu(jnp.dot(msg, w2_ref[...], preferred_element_type=jnp.float32)
                + b2_ref[...])
    bas = jnp.dot(_silu(jnp.dot(msg, vw1_ref[...],
                                preferred_element_type=jnp.float32) + vb1_ref[...]),
                  vw2_ref[...], preferred_element_type=jnp.float32) + vb2_ref[...]
    A = dot(zcat, wa_ref[...])
    B = dot(bas, wb_ref[...])
    za = dot(A * B, wz_ref[...])
    ones = jnp.ones((gs.shape[0], 16), jnp.float32)
    out_ref[...] = jnp.concatenate([msg, za, ones], axis=1)


def _tc_edge(gs, gd, edf, ev, efW, efb, w1in, w1ef,
             w2, b2, vw1, vb1, vw2, vb2, *, be=1280, interpret=False):
    grid = (E // be,)
    sel = _sel_mats()
    row_spec = lambda w: pl.BlockSpec((be, w), lambda i: (i, 0))
    full = lambda a: pl.BlockSpec(a.shape, lambda i: (0,) * a.ndim)
    consts = (efW, efb, w1in, w1ef, w2, b2, vw1, vb1, vw2, vb2) + sel
    return pl.pallas_call(
        _edge_body,
        grid=grid,
        in_specs=[row_spec(SCALAR_IN), row_spec(SCALAR_IN),
                  row_spec(EDGE_IN), row_spec(3)] + [full(c) for c in consts],
        out_specs=row_spec(ROW),
        out_shape=jax.ShapeDtypeStruct((E, ROW), jnp.float32),
        interpret=interpret,
    )(gs, gd, edf, ev, *consts)


def _sc_scatter(eo, dst):
    mesh = plsc.VectorSubcoreMesh(core_axis_name="c", subcore_axis_name="s")

    @functools.partial(
        pl.kernel,
        out_type=jax.ShapeDtypeStruct((NC, N_PAD, ROW), jnp.float32),
        mesh=mesh,
        scratch_types=[
            pltpu.VMEM((CH,), jnp.int32),
            pltpu.VMEM((CH, ROW), jnp.float32),
            pltpu.VMEM((8, ROW), jnp.float32),
            pltpu.VMEM_SHARED((N_PAD, ROW), jnp.float32),
            pltpu.SemaphoreType.DMA,
        ],
    )
    def k(eo_hbm, dst_hbm, part_hbm, idx_v, rows, zb, accum, sem):
        cid = lax.axis_index("c")
        sid = lax.axis_index("s")
        wid = sid * NC + cid

        @pl.loop(0, 8)
        def _(i):
            for j in range(ROW // 16):
                zb[i, pl.ds(j * 16, 16)] = jnp.zeros((16,), jnp.float32)

        @pl.loop(0, NPS // 8)
        def _(t):
            pltpu.sync_copy(zb, accum.at[pl.ds(sid * NPS + t * 8, 8)])

        plsc.subcore_barrier()

        @pl.loop(0, NCHUNK)
        def _(ci):
            base = wid * EW + ci * CH
            pltpu.sync_copy(dst_hbm.at[pl.ds(base, CH)], idx_v)
            pltpu.sync_copy(eo_hbm.at[pl.ds(base, CH)], rows)
            pltpu.sync_copy(rows, accum.at[idx_v], add=True)

        plsc.subcore_barrier()
        pltpu.sync_copy(accum.at[pl.ds(sid * NPS, NPS)],
                        part_hbm.at[cid, pl.ds(sid * NPS, NPS)])

    return k(eo, dst)


def _node_body(p0_ref, p1_ref, h_ref, wh_ref, wm_ref, b1_ref, w2_ref, b2_ref,
               z_ref, h_out_ref):
    p0 = p0_ref[...]
    p1 = p1_ref[...]
    m = p0[:, :HID] + p1[:, :HID]
    zsum = p0[:, HID:HID + 48] + p1[:, HID:HID + 48]
    cnt = p0[:, HID + 48:HID + 49] + p1[:, HID + 48:HID + 49]
    z_ref[...] = zsum / jnp.maximum(cnt, 1.0)
    pre = (jnp.dot(h_ref[...], wh_ref[...], preferred_element_type=jnp.float32)
           + jnp.dot(m, wm_ref[...], preferred_element_type=jnp.float32)
           + b1_ref[...])
    h_out_ref[...] = (jnp.dot(_silu(pre), w2_ref[...],
                              preferred_element_type=jnp.float32) + b2_ref[...])


def _tc_node(p0, p1, h, wh, wm, b1, w2, b2, *, bn=1000, interpret=False):
    grid = (N // bn,)
    row_spec = lambda w: pl.BlockSpec((bn, w), lambda i: (i, 0))
    full = lambda a: pl.BlockSpec(a.shape, lambda i: (0,) * a.ndim)
    return pl.pallas_call(
        _node_body,
        grid=grid,
        in_specs=[row_spec(ROW), row_spec(ROW), row_spec(SCALAR_IN),
                  full(wh), full(wm), full(b1), full(w2), full(b2)],
        out_specs=[row_spec(48), row_spec(SCALAR_IN)],
        out_shape=[jax.ShapeDtypeStruct((N, 48), jnp.float32),
                   jax.ShapeDtypeStruct((N, SCALAR_IN), jnp.float32)],
        interpret=interpret,
    )(p0, p1, h, wh, wm, b1, w2, b2)


def kernel(Z, h, edge_index, edge_distance_feature, edge_distance_vec,
           edge_distance, edge_fc_W, edge_fc_b, msg_W1, msg_b1, msg_W2, msg_b2,
           vec_W1, vec_b1, vec_W2, vec_b2, sc_W1, sc_b1, sc_W2, sc_b2):
    src = edge_index[0]
    dst = edge_index[1]
    zf = Z.reshape(N, 48)

    ts, td = _tc_pre(h, zf, msg_W1[0:128], msg_W1[128:256],
                     msg_b1.reshape(1, HID))
    gs, gd = _sc_gather(ts, td, src, dst)

    eo = _tc_edge(
        gs, gd, edge_distance_feature, edge_distance_vec,
        edge_fc_W, edge_fc_b.reshape(1, HID),
        msg_W1[256:356], msg_W1[356:420],
        msg_W2, msg_b2.reshape(1, HID),
        vec_W1, vec_b1.reshape(1, HID), vec_W2, vec_b2.reshape(1, 80))

    part = _sc_scatter(eo, dst)

    z_out, h_out = _tc_node(
        part[0, :N], part[1, :N], h,
        sc_W1[0:SCALAR_IN], sc_W1[SCALAR_IN:SCALAR_IN + HID],
        sc_b1.reshape(1, HID), sc_W2, sc_b2.reshape(1, SCALAR_OUT))

    return (z_out.reshape(N, 3, VEC_IN), h_out)


# speedup vs baseline: 40.5871x; 1.0119x over previous
import functools

import jax
import jax.numpy as jnp
from jax import lax
from jax.experimental import pallas as pl
from jax.experimental.pallas import tpu as pltpu
from jax.experimental.pallas import tpu_sc as plsc

N = 10000
E = 320000
VEC_IN = 16
SCALAR_IN = 128
EDGE_IN = 16
HID = 64
NH = 4
T = 5
SCALAR_OUT = 128
ROW = 128

NC = 2
NS = 16
NW = NC * NS
EW = E // NW
CH = 80
NCHUNK = EW // CH
NPS = 632
N_PAD = NPS * NS


def _silu(x):
    return x * (1.0 / (1.0 + jnp.exp(-x)))


def _pre_body(h_ref, zf_ref, w1a_ref, w1b_ref, b1_ref, ts_ref, td_ref):
    bn = h_ref.shape[0]
    h = h_ref[...]
    zf = zf_ref[...]
    pad = jnp.zeros((bn, 16), jnp.float32)
    hb = jnp.dot(h, w1b_ref[...], preferred_element_type=jnp.float32)
    ha = jnp.dot(h, w1a_ref[...], preferred_element_type=jnp.float32) + b1_ref[...]
    ts_ref[...] = jnp.concatenate([hb, zf, pad], axis=1)
    td_ref[...] = jnp.concatenate([ha, zf, pad], axis=1)


def _tc_pre(h, zf, w1a, w1b, b1, *, bn=2000, interpret=False):
    row_spec = lambda w: pl.BlockSpec((bn, w), lambda i: (i, 0))
    full = lambda a: pl.BlockSpec(a.shape, lambda i: (0,) * a.ndim)
    return pl.pallas_call(
        _pre_body,
        grid=(N // bn,),
        in_specs=[row_spec(SCALAR_IN), row_spec(48), full(w1a), full(w1b),
                  full(b1)],
        out_specs=[row_spec(SCALAR_IN), row_spec(SCALAR_IN)],
        out_shape=[jax.ShapeDtypeStruct((N, SCALAR_IN), jnp.float32),
                   jax.ShapeDtypeStruct((N, SCALAR_IN), jnp.float32)],
        interpret=interpret,
    )(h, zf, w1a, w1b, b1)


def _sc_gather(ts, td, src, dst):
    mesh = plsc.VectorSubcoreMesh(core_axis_name="c", subcore_axis_name="s")

    @functools.partial(
        pl.kernel,
        out_type=(
            jax.ShapeDtypeStruct((E, SCALAR_IN), jnp.float32),
            jax.ShapeDtypeStruct((E, SCALAR_IN), jnp.float32),
        ),
        mesh=mesh,
        scratch_types=[
            pltpu.VMEM((CH,), jnp.int32),
            pltpu.VMEM((CH,), jnp.int32),
            pltpu.VMEM((CH, SCALAR_IN), jnp.float32),
            pltpu.VMEM((CH, SCALAR_IN), jnp.float32),
            pltpu.SemaphoreType.DMA,
            pltpu.SemaphoreType.DMA,
        ],
    )
    def k(ts_hbm, td_hbm, src_hbm, dst_hbm, gs_hbm, gd_hbm,
          idx_s, idx_d, sb, db, sem1, sem2):
        wid = lax.axis_index("s") * NC + lax.axis_index("c")

        @pl.loop(0, NCHUNK)
        def _(ci):
            base = wid * EW + ci * CH
            pltpu.sync_copy(src_hbm.at[pl.ds(base, CH)], idx_s)
            pltpu.sync_copy(dst_hbm.at[pl.ds(base, CH)], idx_d)
            cp_s = pltpu.async_copy(ts_hbm.at[idx_s], sb, sem1)
            cp_d = pltpu.async_copy(td_hbm.at[idx_d], db, sem2)
            cp_s.wait()
            cp_d.wait()
            pltpu.sync_copy(sb, gs_hbm.at[pl.ds(base, CH)])
            pltpu.sync_copy(db, gd_hbm.at[pl.ds(base, CH)])

    return k(ts, td, src, dst)


def _sel_mats():
    import numpy as np

    def zrow(d, t, h):
        return d * 16 + t * 4 + h if t < 4 else 48 + d

    Wp = np.zeros((51, 300), np.float32)
    Wq = np.zeros((51, 300), np.float32)
    Wr = np.zeros((300, 100), np.float32)
    for d in range(3):
        for t in range(T):
            for r in range(T):
                for h in range(NH):
                    c = d * 100 + (t * T + r) * NH + h
                    Wp[zrow(d, t, h), c] = 1.0
                    Wq[zrow(d, r, h), c] = 1.0
                    Wr[c, (t * T + r) * NH + h] = 1.0
    Wn = np.ones((100, 8), np.float32)
    Wa = np.zeros((51, 240), np.float32)
    Wb = np.zeros((80, 240), np.float32)
    Wz = np.zeros((240, 48), np.float32)
    for d in range(3):
        for t in range(T):
            for k in range(4):
                for h in range(NH):
                    c = d * 80 + t * 16 + k * 4 + h
                    Wa[zrow(d, t, h), c] = 1.0
                    Wb[t * 16 + k * 4 + h, c] = 1.0
                    Wz[c, d * 16 + k * 4 + h] = 1.0
    return (jnp.asarray(Wp, jnp.bfloat16), jnp.asarray(Wq, jnp.bfloat16),
            jnp.asarray(Wr), jnp.asarray(Wn),
            jnp.asarray(Wa, jnp.bfloat16), jnp.asarray(Wb, jnp.bfloat16),
            jnp.asarray(Wz))


def _edge_body(gs_ref, gd_ref, edf_ref, ev_ref,
               efW_ref, efb_ref, w1in_ref, w1ef_ref,
               w2_ref, b2_ref, vw1_ref, vb1_ref, vw2_ref, vb2_ref,
               wp_ref, wq_ref, wr_ref, wn_ref, wa_ref, wb_ref, wz_ref,
               out_ref):
    dot = lambda a, b: jnp.dot(a, b, preferred_element_type=jnp.float32)
    gs = gs_ref[...]
    gd = gd_ref[...]
    zdiff = gd[:, HID:HID + 48] - gs[:, HID:HID + 48]
    zcat = jnp.concatenate([zdiff, ev_ref[...]], axis=1)
    z16 = zcat.astype(jnp.bfloat16)
    P = dot(z16, wp_ref[...])
    Q = dot(z16, wq_ref[...])
    inv = dot(P * Q, wr_ref[...])
    n2 = dot(inv * inv, wn_ref[...])[:, :1]
    inv = inv / jnp.maximum(jnp.sqrt(n2), 1e-12)

    ef = jnp.dot(edf_ref[...], efW_ref[...],
                 preferred_element_type=jnp.float32) + efb_ref[...]
    pre = (gd[:, :HID] + gs[:, :HID]
           + jnp.dot(inv, w1in_ref[...], preferred_element_type=jnp.float32)
           + jnp.dot(ef, w1ef_ref[...], preferred_element_type=jnp.float32))
    msg = _silu(pre)
    msg = _silu(jnp.dot(msg, w2_ref[...], preferred_element_type=jnp.float32)
                + b2_ref[...])
    bas = jnp.dot(_silu(jnp.dot(msg, vw1_ref[...],
                                preferred_element_type=jnp.float32) + vb1_ref[...]),
                  vw2_ref[...], preferred_element_type=jnp.float32) + vb2_ref[...]
    A = dot(z16, wa_ref[...])
    B = dot(bas.astype(jnp.bfloat16), wb_ref[...])
    za = dot(A * B, wz_ref[...])
    ones = jnp.ones((gs.shape[0], 16), jnp.float32)
    out_ref[...] = jnp.concatenate([msg, za, ones], axis=1)


def _tc_edge(gs, gd, edf, ev, efW, efb, w1in, w1ef,
             w2, b2, vw1, vb1, vw2, vb2, *, be=1280, interpret=False):
    grid = (E // be,)
    sel = _sel_mats()
    row_spec = lambda w: pl.BlockSpec((be, w), lambda i: (i, 0))
    full = lambda a: pl.BlockSpec(a.shape, lambda i: (0,) * a.ndim)
    consts = (efW, efb, w1in, w1ef, w2, b2, vw1, vb1, vw2, vb2) + sel
    return pl.pallas_call(
        _edge_body,
        grid=grid,
        in_specs=[row_spec(SCALAR_IN), row_spec(SCALAR_IN),
                  row_spec(EDGE_IN), row_spec(3)] + [full(c) for c in consts],
        out_specs=row_spec(ROW),
        out_shape=jax.ShapeDtypeStruct((E, ROW), jnp.float32),
        interpret=interpret,
    )(gs, gd, edf, ev, *consts)


def _sc_scatter(eo, dst):
    mesh = plsc.VectorSubcoreMesh(core_axis_name="c", subcore_axis_name="s")

    @functools.partial(
        pl.kernel,
        out_type=jax.ShapeDtypeStruct((NC, N_PAD, ROW), jnp.float32),
        mesh=mesh,
        scratch_types=[
            pltpu.VMEM((CH,), jnp.int32),
            pltpu.VMEM((CH, ROW), jnp.float32),
            pltpu.VMEM((8, ROW), jnp.float32),
            pltpu.VMEM_SHARED((N_PAD, ROW), jnp.float32),
            pltpu.SemaphoreType.DMA,
        ],
    )
    def k(eo_hbm, dst_hbm, part_hbm, idx_v, rows, zb, accum, sem):
        cid = lax.axis_index("c")
        sid = lax.axis_index("s")
        wid = sid * NC + cid

        @pl.loop(0, 8)
        def _(i):
            for j in range(ROW // 16):
                zb[i, pl.ds(j * 16, 16)] = jnp.zeros((16,), jnp.float32)

        @pl.loop(0, NPS // 8)
        def _(t):
            pltpu.sync_copy(zb, accum.at[pl.ds(sid * NPS + t * 8, 8)])

        plsc.subcore_barrier()

        @pl.loop(0, NCHUNK)
        def _(ci):
            base = wid * EW + ci * CH
            pltpu.sync_copy(dst_hbm.at[pl.ds(base, CH)], idx_v)
            pltpu.sync_copy(eo_hbm.at[pl.ds(base, CH)], rows)
            pltpu.sync_copy(rows, accum.at[idx_v], add=True)

        plsc.subcore_barrier()
        pltpu.sync_copy(accum.at[pl.ds(sid * NPS, NPS)],
                        part_hbm.at[cid, pl.ds(sid * NPS, NPS)])

    return k(eo, dst)


def _node_body(p0_ref, p1_ref, h_ref, wh_ref, wm_ref, b1_ref, w2_ref, b2_ref,
               z_ref, h_out_ref):
    p0 = p0_ref[...]
    p1 = p1_ref[...]
    m = p0[:, :HID] + p1[:, :HID]
    zsum = p0[:, HID:HID + 48] + p1[:, HID:HID + 48]
    cnt = p0[:, HID + 48:HID + 49] + p1[:, HID + 48:HID + 49]
    z_ref[...] = zsum / jnp.maximum(cnt, 1.0)
    pre = (jnp.dot(h_ref[...], wh_ref[...], preferred_element_type=jnp.float32)
           + jnp.dot(m, wm_ref[...], preferred_element_type=jnp.float32)
           + b1_ref[...])
    h_out_ref[...] = (jnp.dot(_silu(pre), w2_ref[...],
                              preferred_element_type=jnp.float32) + b2_ref[...])


def _tc_node(p0, p1, h, wh, wm, b1, w2, b2, *, bn=1000, interpret=False):
    grid = (N // bn,)
    row_spec = lambda w: pl.BlockSpec((bn, w), lambda i: (i, 0))
    full = lambda a: pl.BlockSpec(a.shape, lambda i: (0,) * a.ndim)
    return pl.pallas_call(
        _node_body,
        grid=grid,
        in_specs=[row_spec(ROW), row_spec(ROW), row_spec(SCALAR_IN),
                  full(wh), full(wm), full(b1), full(w2), full(b2)],
        out_specs=[row_spec(48), row_spec(SCALAR_IN)],
        out_shape=[jax.ShapeDtypeStruct((N, 48), jnp.float32),
                   jax.ShapeDtypeStruct((N, SCALAR_IN), jnp.float32)],
        interpret=interpret,
    )(p0, p1, h, wh, wm, b1, w2, b2)


def kernel(Z, h, edge_index, edge_distance_feature, edge_distance_vec,
           edge_distance, edge_fc_W, edge_fc_b, msg_W1, msg_b1, msg_W2, msg_b2,
           vec_W1, vec_b1, vec_W2, vec_b2, sc_W1, sc_b1, sc_W2, sc_b2):
    src = edge_index[0]
    dst = edge_index[1]
    zf = Z.reshape(N, 48)

    ts, td = _tc_pre(h, zf, msg_W1[0:128], msg_W1[128:256],
                     msg_b1.reshape(1, HID))
    gs, gd = _sc_gather(ts, td, src, dst)

    eo = _tc_edge(
        gs, gd, edge_distance_feature, edge_distance_vec,
        edge_fc_W, edge_fc_b.reshape(1, HID),
        msg_W1[256:356], msg_W1[356:420],
        msg_W2, msg_b2.reshape(1, HID),
        vec_W1, vec_b1.reshape(1, HID), vec_W2, vec_b2.reshape(1, 80))

    part = _sc_scatter(eo, dst)

    z_out, h_out = _tc_node(
        part[0, :N], part[1, :N], h,
        sc_W1[0:SCALAR_IN], sc_W1[SCALAR_IN:SCALAR_IN + HID],
        sc_b1.reshape(1, HID), sc_W2, sc_b2.reshape(1, SCALAR_OUT))

    return (z_out.reshape(N, 3, VEC_IN), h_out)


# speedup vs baseline: 53.0442x; 1.3069x over previous
import functools

import jax
import jax.numpy as jnp
from jax import lax
from jax.experimental import pallas as pl
from jax.experimental.pallas import tpu as pltpu
from jax.experimental.pallas import tpu_sc as plsc

N = 10000
E = 320000
VEC_IN = 16
SCALAR_IN = 128
EDGE_IN = 16
HID = 64
NH = 4
T = 5
SCALAR_OUT = 128
ROW = 128

NC = 2
NS = 16
NW = NC * NS
CH = 80
NSLICE = 5
ES = E // NSLICE
EWS = ES // NW
NCHS = EWS // CH
NPS = 632
N_PAD = NPS * NS


def _silu(x):
    return x * (1.0 / (1.0 + jnp.exp(-x)))


def _pre_body(h_ref, zf_ref, w1a_ref, w1b_ref, b1_ref, ts_ref, td_ref):
    bn = h_ref.shape[0]
    h = h_ref[...]
    zf = zf_ref[...]
    pad = jnp.zeros((bn, 16), jnp.float32)
    hb = jnp.dot(h, w1b_ref[...], preferred_element_type=jnp.float32)
    ha = jnp.dot(h, w1a_ref[...], preferred_element_type=jnp.float32) + b1_ref[...]
    ts_ref[...] = jnp.concatenate([hb, zf, pad], axis=1)
    td_ref[...] = jnp.concatenate([ha, zf, pad], axis=1)


def _tc_pre(h, zf, w1a, w1b, b1, *, bn=2000, interpret=False):
    row_spec = lambda w: pl.BlockSpec((bn, w), lambda i: (i, 0))
    full = lambda a: pl.BlockSpec(a.shape, lambda i: (0,) * a.ndim)
    return pl.pallas_call(
        _pre_body,
        grid=(N // bn,),
        in_specs=[row_spec(SCALAR_IN), row_spec(48), full(w1a), full(w1b),
                  full(b1)],
        out_specs=[row_spec(SCALAR_IN), row_spec(SCALAR_IN)],
        out_shape=[jax.ShapeDtypeStruct((N, SCALAR_IN), jnp.float32),
                   jax.ShapeDtypeStruct((N, SCALAR_IN), jnp.float32)],
        interpret=interpret,
    )(h, zf, w1a, w1b, b1)


def _sc_gather(ts, td, src, dst):
    mesh = plsc.VectorSubcoreMesh(core_axis_name="c", subcore_axis_name="s")

    @functools.partial(
        pl.kernel,
        out_type=(
            jax.ShapeDtypeStruct((ES, SCALAR_IN), jnp.float32),
            jax.ShapeDtypeStruct((ES, SCALAR_IN), jnp.float32),
        ),
        mesh=mesh,
        scratch_types=[
            pltpu.VMEM((CH,), jnp.int32),
            pltpu.VMEM((CH,), jnp.int32),
            pltpu.VMEM((CH, SCALAR_IN), jnp.float32),
            pltpu.VMEM((CH, SCALAR_IN), jnp.float32),
            pltpu.SemaphoreType.DMA,
            pltpu.SemaphoreType.DMA,
        ],
    )
    def k(ts_hbm, td_hbm, src_hbm, dst_hbm, gs_hbm, gd_hbm,
          idx_s, idx_d, sb, db, sem1, sem2):
        wid = lax.axis_index("s") * NC + lax.axis_index("c")

        @pl.loop(0, NCHS)
        def _(ci):
            base = wid * EWS + ci * CH
            pltpu.sync_copy(src_hbm.at[pl.ds(base, CH)], idx_s)
            pltpu.sync_copy(dst_hbm.at[pl.ds(base, CH)], idx_d)
            cp_s = pltpu.async_copy(ts_hbm.at[idx_s], sb, sem1)
            cp_d = pltpu.async_copy(td_hbm.at[idx_d], db, sem2)
            cp_s.wait()
            cp_d.wait()
            pltpu.sync_copy(sb, gs_hbm.at[pl.ds(base, CH)])
            pltpu.sync_copy(db, gd_hbm.at[pl.ds(base, CH)])

    return k(ts, td, src, dst)


def _sel_mats():
    import numpy as np

    def zrow(d, t, h):
        return d * 16 + t * 4 + h if t < 4 else 48 + d

    Wp = np.zeros((51, 300), np.float32)
    Wq = np.zeros((51, 300), np.float32)
    Wr = np.zeros((300, 100), np.float32)
    for d in range(3):
        for t in range(T):
            for r in range(T):
                for h in range(NH):
                    c = d * 100 + (t * T + r) * NH + h
                    Wp[zrow(d, t, h), c] = 1.0
                    Wq[zrow(d, r, h), c] = 1.0
                    Wr[c, (t * T + r) * NH + h] = 1.0
    Wn = np.ones((100, 8), np.float32)
    Wa = np.zeros((51, 240), np.float32)
    Wb = np.zeros((80, 240), np.float32)
    Wz = np.zeros((240, 48), np.float32)
    for d in range(3):
        for t in range(T):
            for k in range(4):
                for h in range(NH):
                    c = d * 80 + t * 16 + k * 4 + h
                    Wa[zrow(d, t, h), c] = 1.0
                    Wb[t * 16 + k * 4 + h, c] = 1.0
                    Wz[c, d * 16 + k * 4 + h] = 1.0
    return (jnp.asarray(Wp, jnp.bfloat16), jnp.asarray(Wq, jnp.bfloat16),
            jnp.asarray(Wr), jnp.asarray(Wn),
            jnp.asarray(Wa, jnp.bfloat16), jnp.asarray(Wb, jnp.bfloat16),
            jnp.asarray(Wz))


def _edge_body(gs_ref, gd_ref, edf_ref, ev_ref,
               efW_ref, efb_ref, w1in_ref, w1ef_ref,
               w2_ref, b2_ref, vw1_ref, vb1_ref, vw2_ref, vb2_ref,
               wp_ref, wq_ref, wr_ref, wn_ref, wa_ref, wb_ref, wz_ref,
               out_ref):
    dot = lambda a, b: jnp.dot(a, b, preferred_element_type=jnp.float32)
    gs = gs_ref[...]
    gd = gd_ref[...]
    zdiff = gd[:, HID:HID + 48] - gs[:, HID:HID + 48]
    zcat = jnp.concatenate([zdiff, ev_ref[...]], axis=1)
    z16 = zcat.astype(jnp.bfloat16)
    P = dot(z16, wp_ref[...])
    Q = dot(z16, wq_ref[...])
    inv = dot(P * Q, wr_ref[...])
    n2 = dot(inv * inv, wn_ref[...])[:, :1]
    inv = inv / jnp.maximum(jnp.sqrt(n2), 1e-12)

    ef = jnp.dot(edf_ref[...], efW_ref[...],
                 preferred_element_type=jnp.float32) + efb_ref[...]
    pre = (gd[:, :HID] + gs[:, :HID]
           + jnp.dot(inv, w1in_ref[...], preferred_element_type=jnp.float32)
           + jnp.dot(ef, w1ef_ref[...], preferred_element_type=jnp.float32))
    msg = _silu(pre)
    msg = _silu(jnp.dot(msg, w2_ref[...], preferred_element_type=jnp.float32)
                + b2_ref[...])
    bas = jnp.dot(_silu(jnp.dot(msg, vw1_ref[...],
                                preferred_element_type=jnp.float32) + vb1_ref[...]),
                  vw2_ref[...], preferred_element_type=jnp.float32) + vb2_ref[...]
    A = dot(z16, wa_ref[...])
    B = dot(bas.astype(jnp.bfloat16), wb_ref[...])
    za = dot(A * B, wz_ref[...])
    ones = jnp.ones((gs.shape[0], 16), jnp.float32)
    out_ref[...] = jnp.concatenate([msg, za, ones], axis=1)


def _tc_edge(gs, gd, edf, ev, efW, efb, w1in, w1ef,
             w2, b2, vw1, vb1, vw2, vb2, *, be=1280, interpret=False):
    ne = gs.shape[0]
    grid = (ne // be,)
    sel = _sel_mats()
    row_spec = lambda w: pl.BlockSpec((be, w), lambda i: (i, 0))
    full = lambda a: pl.BlockSpec(a.shape, lambda i: (0,) * a.ndim)
    consts = (efW, efb, w1in, w1ef, w2, b2, vw1, vb1, vw2, vb2) + sel
    return pl.pallas_call(
        _edge_body,
        grid=grid,
        in_specs=[row_spec(SCALAR_IN), row_spec(SCALAR_IN),
                  row_spec(EDGE_IN), row_spec(3)] + [full(c) for c in consts],
        out_specs=row_spec(ROW),
        out_shape=jax.ShapeDtypeStruct((ne, ROW), jnp.float32),
        interpret=interpret,
    )(gs, gd, edf, ev, *consts)


def _sc_scatter(eo, dst):
    mesh = plsc.VectorSubcoreMesh(core_axis_name="c", subcore_axis_name="s")

    @functools.partial(
        pl.kernel,
        out_type=jax.ShapeDtypeStruct((NC, N_PAD, ROW), jnp.float32),
        mesh=mesh,
        scratch_types=[
            pltpu.VMEM((CH,), jnp.int32),
            pltpu.VMEM((CH, ROW), jnp.float32),
            pltpu.VMEM((8, ROW), jnp.float32),
            pltpu.VMEM_SHARED((N_PAD, ROW), jnp.float32),
            pltpu.SemaphoreType.DMA,
        ],
    )
    def k(eo_hbm, dst_hbm, part_hbm, idx_v, rows, zb, accum, sem):
        cid = lax.axis_index("c")
        sid = lax.axis_index("s")
        wid = sid * NC + cid

        @pl.loop(0, 8)
        def _(i):
            for j in range(ROW // 16):
                zb[i, pl.ds(j * 16, 16)] = jnp.zeros((16,), jnp.float32)

        @pl.loop(0, NPS // 8)
        def _(t):
            pltpu.sync_copy(zb, accum.at[pl.ds(sid * NPS + t * 8, 8)])

        plsc.subcore_barrier()

        @pl.loop(0, NCHS)
        def _(ci):
            base = wid * EWS + ci * CH
            pltpu.sync_copy(dst_hbm.at[pl.ds(base, CH)], idx_v)
            pltpu.sync_copy(eo_hbm.at[pl.ds(base, CH)], rows)
            pltpu.sync_copy(rows, accum.at[idx_v], add=True)

        plsc.subcore_barrier()
        pltpu.sync_copy(accum.at[pl.ds(sid * NPS, NPS)],
                        part_hbm.at[cid, pl.ds(sid * NPS, NPS)])

    return k(eo, dst)


def _node_body(*refs):
    part_refs = refs[:-8]
    h_ref, wh_ref, wm_ref, b1_ref, w2_ref, b2_ref, z_ref, h_out_ref = refs[-8:]
    acc = None
    for pr in part_refs:
        pall = pr[...]
        for c in range(NC):
            p = pall[c]
            acc = p if acc is None else acc + p
    m = acc[:, :HID]
    zsum = acc[:, HID:HID + 48]
    cnt = acc[:, HID + 48:HID + 49]
    z_ref[...] = zsum / jnp.maximum(cnt, 1.0)
    pre = (jnp.dot(h_ref[...], wh_ref[...], preferred_element_type=jnp.float32)
           + jnp.dot(m, wm_ref[...], preferred_element_type=jnp.float32)
           + b1_ref[...])
    h_out_ref[...] = (jnp.dot(_silu(pre), w2_ref[...],
                              preferred_element_type=jnp.float32) + b2_ref[...])


def _tc_node(parts, h, wh, wm, b1, w2, b2, *, bn=1000, interpret=False):
    grid = (N // bn,)
    row_spec = lambda w: pl.BlockSpec((bn, w), lambda i: (i, 0))
    part_spec = pl.BlockSpec((NC, bn, ROW), lambda i: (0, i, 0))
    full = lambda a: pl.BlockSpec(a.shape, lambda i: (0,) * a.ndim)
    return pl.pallas_call(
        _node_body,
        grid=grid,
        in_specs=[part_spec] * len(parts)
        + [row_spec(SCALAR_IN), full(wh), full(wm), full(b1), full(w2),
           full(b2)],
        out_specs=[row_spec(48), row_spec(SCALAR_IN)],
        out_shape=[jax.ShapeDtypeStruct((N, 48), jnp.float32),
                   jax.ShapeDtypeStruct((N, SCALAR_IN), jnp.float32)],
        interpret=interpret,
    )(*parts, h, wh, wm, b1, w2, b2)


def kernel(Z, h, edge_index, edge_distance_feature, edge_distance_vec,
           edge_distance, edge_fc_W, edge_fc_b, msg_W1, msg_b1, msg_W2, msg_b2,
           vec_W1, vec_b1, vec_W2, vec_b2, sc_W1, sc_b1, sc_W2, sc_b2):
    src = edge_index[0]
    dst = edge_index[1]
    zf = Z.reshape(N, 48)

    ts, td = _tc_pre(h, zf, msg_W1[0:128], msg_W1[128:256],
                     msg_b1.reshape(1, HID))

    parts = []
    for s in range(NSLICE):
        lo, hi = s * ES, (s + 1) * ES
        src_s, dst_s = src[lo:hi], dst[lo:hi]
        gs, gd = _sc_gather(ts, td, src_s, dst_s)
        eo = _tc_edge(
            gs, gd, edge_distance_feature[lo:hi], edge_distance_vec[lo:hi],
            edge_fc_W, edge_fc_b.reshape(1, HID),
            msg_W1[256:356], msg_W1[356:420],
            msg_W2, msg_b2.reshape(1, HID),
            vec_W1, vec_b1.reshape(1, HID), vec_W2, vec_b2.reshape(1, 80))
        parts.append(_sc_scatter(eo, dst_s))

    z_out, h_out = _tc_node(
        parts, h,
        sc_W1[0:SCALAR_IN], sc_W1[SCALAR_IN:SCALAR_IN + HID],
        sc_b1.reshape(1, HID), sc_W2, sc_b2.reshape(1, SCALAR_OUT))

    return (z_out.reshape(N, 3, VEC_IN), h_out)
